# Initial kernel scaffold; baseline (speedup 1.0000x reference)
import jax
import jax.numpy as jnp
from jax.experimental import pallas as pl


def kernel(x, edge_index, edge_attr, batch, W1, as1, ad1, We1, ae1, b1, W2, as2, ad2, We2, ae2, b2, W3, as3, ad3, We3, ae3, b3, g1, be1, g2, be2, g3, be3, pw1, pw2, pw3, fW1, fb1, fW2, fb2):
    raise NotImplementedError("write your pallas kernel here")


# speedup vs baseline: 1.0000x; 1.0000x over previous
import jax
import jax.numpy as jnp
from jax.experimental import pallas as pl
from jax.experimental.pallas import tpu as pltpu

N = 50000
E = 800000
G = 512
RATIO = 0.9


def _gat_j(x, ei, ea, W, a_s, a_d, We, a_e, b, emask):
    Nn = x.shape[0]
    src = ei[0]
    dst = ei[1]
    h = (x @ W).reshape(Nn, 2, 64)
    e = (ea @ We).reshape(-1, 2, 64)
    al = (h * a_s).sum(-1)[src] + (h * a_d).sum(-1)[dst] + (e * a_e).sum(-1)
    al = jnp.where(al > 0, al, 0.2 * al)
    al = jnp.where(emask[:, None], al, -jnp.inf)
    amax = jax.ops.segment_max(al, dst, num_segments=Nn)
    amax = jnp.where(jnp.isfinite(amax), amax, 0.0)
    al = al - jax.lax.stop_gradient(amax)[dst]
    ex = jnp.exp(al)
    den = jax.ops.segment_sum(ex, dst, num_segments=Nn)[dst] + 1e-16
    coef = ex / den
    out = jax.ops.segment_sum(h[src] * coef[:, :, None], dst, num_segments=Nn)
    return out.reshape(Nn, 128) + b


def _bn_j(x, g, b, nmask):
    w = nmask.astype(x.dtype)
    cnt = w.sum()
    m = (x * w[:, None]).sum(0) / cnt
    d = x - m
    v = (d * d * w[:, None]).sum(0) / cnt
    return g * (x - m) / jnp.sqrt(v + 1e-5) + b


def _gmp_j(x, bt, nmask):
    w = nmask.astype(x.dtype)
    s = jax.ops.segment_sum(x * w[:, None], bt, num_segments=G)
    c = jax.ops.segment_sum(w, bt, num_segments=G)
    return s / jnp.maximum(c, 1.0)[:, None]


def _pool_j(x, ei, ea, bt, w, nmask, emask):
    score = jnp.tanh((x @ w) / (jnp.linalg.norm(w) + 1e-16))
    Nn = x.shape[0]
    vi = nmask.astype(jnp.int32)
    counts = jax.ops.segment_sum(vi, bt, num_segments=G)
    starts = jnp.concatenate([jnp.zeros((1,), counts.dtype), jnp.cumsum(counts)[:-1]])
    order = jnp.lexsort((-score, bt, 1 - vi))
    bs = bt[order]
    rank = jnp.arange(Nn, dtype=counts.dtype) - starts[bs]
    kper = jnp.ceil(RATIO * counts).astype(counts.dtype)
    keep = (rank < kper[bs]) & nmask[order]
    perm = order[jnp.argsort(~keep)]
    kept = jnp.cumsum(keep)
    new_nmask = jnp.arange(Nn) < kept[-1]
    pos = jnp.where(keep, (kept - 1).astype(ei.dtype), 0)
    new_idx = jnp.zeros((Nn,), dtype=ei.dtype).at[order].set(pos)
    mask_old = jnp.zeros((Nn,), bool).at[order].set(keep)
    new_emask = emask & mask_old[ei[0]] & mask_old[ei[1]]
    new_ei = jnp.where(new_emask[None, :], new_idx[ei], 0)
    new_bt = jnp.where(new_nmask, bt[perm], 0)
    xn = (x[perm] * score[perm][:, None]) * new_nmask[:, None].astype(x.dtype)
    return xn, new_ei, ea, new_bt, new_nmask, new_emask


def _mlp_body(z_ref, fW1_ref, fb1_ref, fW2_ref, fb2_ref, o_ref):
    z = z_ref[...]
    h = jnp.maximum(z @ fW1_ref[...] + fb1_ref[...][None, :], 0.0)
    o_ref[...] = h @ fW2_ref[...] + fb2_ref[...][None, :]


def _mlp_head(z, fW1, fb1, fW2, fb2):
    return pl.pallas_call(
        _mlp_body,
        out_shape=jax.ShapeDtypeStruct((G, 1), jnp.float32),
    )(z, fW1, fb1, fW2, fb2)


def kernel(x, edge_index, edge_attr, batch, W1, as1, ad1, We1, ae1, b1, W2, as2, ad2, We2, ae2, b2, W3, as3, ad3, We3, ae3, b3, g1, be1, g2, be2, g3, be3, pw1, pw2, pw3, fW1, fb1, fW2, fb2):
    h, ei, ea, bt = x, edge_index, edge_attr, batch
    nmask = jnp.ones((x.shape[0],), bool)
    emask = jnp.ones((edge_index.shape[1],), bool)

    h = jax.nn.relu(_bn_j(_gat_j(h, ei, ea, W1, as1, ad1, We1, ae1, b1, emask), g1, be1, nmask))
    h, ei, ea, bt, nmask, emask = _pool_j(h, ei, ea, bt, pw1, nmask, emask)
    r1 = _gmp_j(h, bt, nmask)

    h = jax.nn.relu(_bn_j(_gat_j(h, ei, ea, W2, as2, ad2, We2, ae2, b2, emask), g2, be2, nmask))
    h, ei, ea, bt, nmask, emask = _pool_j(h, ei, ea, bt, pw2, nmask, emask)
    r2 = _gmp_j(h, bt, nmask)

    h = jax.nn.relu(_bn_j(_gat_j(h, ei, ea, W3, as3, ad3, We3, ae3, b3, emask), g3, be3, nmask))
    h, ei, ea, bt, nmask, emask = _pool_j(h, ei, ea, bt, pw3, nmask, emask)
    r3 = _gmp_j(h, bt, nmask)

    z = r1 + r2 + r3
    return _mlp_head(z, fW1, fb1, fW2, fb2)


# speedup vs baseline: 6.2166x; 6.2166x over previous
import dataclasses
import functools

import jax
import jax.numpy as jnp
from jax import lax
from jax.experimental import pallas as pl
from jax.experimental.pallas import tpu as pltpu
from jax.experimental.pallas import tpu_sc as plsc

N = 50000
E = 800000
G = 512
RATIO = 0.9

NC = 2
NS = 16
L = 16

EB = 2048
NB = 25
EPT = EB * NB
E_PAD = NS * EPT

GBR = 256
WCAP = 288
CHUNK = 10240
NCHUNK = 6
NPAD = NCHUNK * CHUNK
DEN_PAD = NS * 3136

_vmesh = plsc.VectorSubcoreMesh(core_axis_name="c", subcore_axis_name="s")

_sc_params = pltpu.CompilerParams()
if "needs_layout_passes" in pltpu.CompilerParams.__dataclass_fields__:
    _sc_params = dataclasses.replace(_sc_params, needs_layout_passes=False)


def _splat(val, n=L):
    return jnp.broadcast_to(val, (n,))


EBR = EB // 128
RPT = EPT // 128
EP2 = E_PAD // 128


@functools.partial(
    pl.kernel,
    out_type=jax.ShapeDtypeStruct((2 * EP2, 128), jnp.float32),
    mesh=_vmesh,
    compiler_params=_sc_params,
    scratch_types=[
        pltpu.VMEM((N,), jnp.float32),
        pltpu.VMEM((N,), jnp.float32),
        pltpu.VMEM((EBR, 128), jnp.int32),
        pltpu.VMEM((EBR, 128), jnp.int32),
        pltpu.VMEM((EBR, 128), jnp.float32),
        pltpu.VMEM((EBR, 128), jnp.float32),
        pltpu.SemaphoreType.DMA,
        pltpu.SemaphoreType.DMA,
        pltpu.SemaphoreType.DMA,
        pltpu.SemaphoreType.DMA,
    ],
)
def _edge_logit_kernel(src_hbm, dst_hbm, eal_hbm, s_hbm, d_hbm,
                       al_hbm,
                       s_vm, d_vm, srcb, dstb, ealb, alb,
                       sem0, sem1, sem2, sem3):
    c = lax.axis_index("c")
    s = lax.axis_index("s")

    cp_s = pltpu.async_copy(s_hbm.at[pl.ds(c * N, N)], s_vm, sem0)
    cp_d = pltpu.async_copy(d_hbm.at[pl.ds(c * N, N)], d_vm, sem1)
    cp_s.wait()
    cp_d.wait()

    def batch(b):
        rbase = s * RPT + b * EBR
        c1 = pltpu.async_copy(src_hbm.at[pl.ds(rbase, EBR)], srcb, sem0)
        c2 = pltpu.async_copy(dst_hbm.at[pl.ds(rbase, EBR)], dstb, sem1)
        c3 = pltpu.async_copy(eal_hbm.at[pl.ds(c * EP2 + rbase, EBR)], ealb, sem2)
        c1.wait(); c2.wait(); c3.wait()

        @pl.loop(0, EBR)
        def _(jr):
            @pl.loop(0, 128, step=L)
            def _(jc):
                sl = pl.ds(jc, L)
                sv = plsc.load_gather(s_vm, [srcb[jr, sl]])
                dv = plsc.load_gather(d_vm, [dstb[jr, sl]])
                al = sv + dv + ealb[jr, sl]
                alb[jr, sl] = jnp.maximum(al, 0.2 * al)

        pltpu.async_copy(alb, al_hbm.at[pl.ds(c * EP2 + rbase, EBR)], sem3).wait()

    @pl.loop(0, NB)
    def _(b):
        batch(b)


@functools.partial(
    pl.kernel,
    out_type=[
        jax.ShapeDtypeStruct((NPAD, 128), jnp.float32),
        jax.ShapeDtypeStruct((2 * NPAD,), jnp.float32),
    ],
    mesh=_vmesh,
    compiler_params=_sc_params,
    scratch_types=[
        pltpu.VMEM((EBR, 128), jnp.int32),
        pltpu.VMEM((EBR, 128), jnp.int32),
        pltpu.VMEM((EBR, 128), jnp.float32),
        pltpu.VMEM((EBR, 128), jnp.float32),
        pltpu.VMEM((WCAP,), jnp.int32),
        pltpu.VMEM((WCAP,), jnp.int32),
        pltpu.VMEM((WCAP,), jnp.float32),
        pltpu.VMEM((WCAP,), jnp.float32),
        pltpu.VMEM((2, 128), jnp.int32),
        pltpu.VMEM((2, 128), jnp.int32),
        pltpu.VMEM((2, 128), jnp.float32),
        pltpu.VMEM((2, 128), jnp.float32),
        pltpu.VMEM((GBR, 128), jnp.float32),
        pltpu.VMEM((640,), jnp.float32),
        pltpu.SMEM((1,), jnp.int32),
        pltpu.VMEM_SHARED((CHUNK, 128), jnp.float32),
        pltpu.VMEM_SHARED((CHUNK,), jnp.float32),
        pltpu.VMEM_SHARED((CHUNK,), jnp.float32),
        pltpu.SemaphoreType.DMA,
        pltpu.SemaphoreType.DMA,
        pltpu.SemaphoreType.DMA,
        pltpu.SemaphoreType.DMA,
        pltpu.SemaphoreType.DMA,
    ],
)
def _aggregate_kernel(src_hbm, dst_hbm, ex0_hbm, ex1_hbm, h_hbm,
                      out_hbm, den_hbm,
                      srcb, dstb, e0b, e1b, wsrc, wdst, ww0, ww1,
                      gsrc, gdst, gw0, gw1, rows, dzbuf, cnt_ref,
                      acc_sh, den0_sh, den1_sh,
                      sem0, sem1, sem2, sem3, semg):
    c = lax.axis_index("c")
    s = lax.axis_index("s")

    def drain():
        for j in range(2):
            @pl.loop(0, 128, step=L)
            def _(i):
                gsrc[j, pl.ds(i, L)] = wsrc[pl.ds(j * 128 + i, L)]
                gdst[j, pl.ds(i, L)] = wdst[pl.ds(j * 128 + i, L)]
                gw0[j, pl.ds(i, L)] = ww0[pl.ds(j * 128 + i, L)]
                gw1[j, pl.ds(i, L)] = ww1[pl.ds(j * 128 + i, L)]

        cps = [pltpu.async_copy(h_hbm.at[gsrc.at[j]],
                                rows.at[pl.ds(j * 128, 128)], semg)
               for j in range(2)]
        for cp in cps:
            cp.wait()

        @pl.loop(0, GBR)
        def _(r):
            ridx = _splat(r)
            w0 = plsc.load_gather(ww0, [ridx])
            w1 = plsc.load_gather(ww1, [ridx])
            for cc in range(4):
                sl = pl.ds(cc * L, L)
                rows[r, sl] = rows[r, sl] * w0
            for cc in range(4, 8):
                sl = pl.ds(cc * L, L)
                rows[r, sl] = rows[r, sl] * w1

        for j in range(2):
            pltpu.sync_copy(rows.at[pl.ds(j * 128, 128)],
                            acc_sh.at[gdst.at[j]], add=True)
            pltpu.sync_copy(gw0.at[j], den0_sh.at[gdst.at[j]], add=True)
            pltpu.sync_copy(gw1.at[j], den1_sh.at[gdst.at[j]], add=True)
        cnt = cnt_ref[0]
        wsrc[pl.ds(0, L)] = wsrc[pl.ds(GBR, L)]
        wdst[pl.ds(0, L)] = wdst[pl.ds(GBR, L)]
        ww0[pl.ds(0, L)] = ww0[pl.ds(GBR, L)]
        ww1[pl.ds(0, L)] = ww1[pl.ds(GBR, L)]
        cnt_ref[0] = jnp.maximum(cnt - GBR, 0)

    for p in range(3):
        chunk = 2 * p + c
        lo = chunk * CHUNK

        @pl.loop(0, GBR)
        def _(r):
            @pl.loop(0, 128, step=L)
            def _(j):
                rows[r, pl.ds(j, L)] = jnp.zeros((L,), jnp.float32)

        for off, nrow in ((0, 256), (256, 256), (512, 128)):
            pltpu.sync_copy(rows.at[pl.ds(0, nrow)],
                            acc_sh.at[pl.ds(s * 640 + off, nrow)])
        for k in range(5):
            pltpu.sync_copy(rows.at[0], den0_sh.at[pl.ds(s * 640 + k * 128, 128)])
            pltpu.sync_copy(rows.at[0], den1_sh.at[pl.ds(s * 640 + k * 128, 128)])

        @pl.loop(0, WCAP, step=L)
        def _(i):
            wsrc[pl.ds(i, L)] = jnp.zeros((L,), jnp.int32)
            wdst[pl.ds(i, L)] = jnp.zeros((L,), jnp.int32)
            ww0[pl.ds(i, L)] = jnp.zeros((L,), jnp.float32)
            ww1[pl.ds(i, L)] = jnp.zeros((L,), jnp.float32)

        cnt_ref[0] = 0
        plsc.subcore_barrier()

        def batch(b):
            rbase = s * RPT + b * EBR
            c1 = pltpu.async_copy(src_hbm.at[pl.ds(rbase, EBR)], srcb, sem0)
            c2 = pltpu.async_copy(dst_hbm.at[pl.ds(rbase, EBR)], dstb, sem1)
            c3 = pltpu.async_copy(ex0_hbm.at[pl.ds(rbase, EBR)], e0b, sem2)
            c4 = pltpu.async_copy(ex1_hbm.at[pl.ds(rbase, EBR)], e1b, sem3)
            c1.wait(); c2.wait(); c3.wait(); c4.wait()

            @pl.loop(0, EBR)
            def _(jr):
                @pl.loop(0, 128, step=L)
                def _(jc):
                    sl = pl.ds(jc, L)
                    dv = dstb[jr, sl]
                    dl = dv - lo
                    w0 = e0b[jr, sl]
                    w1 = e1b[jr, sl]
                    m = (dl >= 0) & (dl < CHUNK) & ((w0 + w1) > 0.0)
                    cnt = cnt_ref[0]
                    plsc.store_compressed(wsrc.at[pl.ds(cnt, L)], srcb[jr, sl], mask=m)
                    plsc.store_compressed(wdst.at[pl.ds(cnt, L)], dl, mask=m)
                    plsc.store_compressed(ww0.at[pl.ds(cnt, L)], w0, mask=m)
                    plsc.store_compressed(ww1.at[pl.ds(cnt, L)], w1, mask=m)
                    cnt_ref[0] = cnt + jnp.sum(m.astype(jnp.int32))

                    @pl.when(cnt_ref[0] >= GBR)
                    def _():
                        drain()

        @pl.loop(0, NB)
        def _(b):
            batch(b)

        cnt_tail = cnt_ref[0]

        @pl.loop(0, WCAP, step=L)
        def _(i):
            lane = lax.iota(jnp.int32, L) + i
            dead = lane >= cnt_tail
            ww0[pl.ds(i, L)] = jnp.where(dead, 0.0, ww0[pl.ds(i, L)])
            ww1[pl.ds(i, L)] = jnp.where(dead, 0.0, ww1[pl.ds(i, L)])

        drain()
        cnt_ref[0] = 0
        plsc.subcore_barrier()

        for off, nrow in ((0, 256), (256, 256), (512, 128)):
            pltpu.sync_copy(acc_sh.at[pl.ds(s * 640 + off, nrow)],
                            rows.at[pl.ds(0, nrow)])
            pltpu.sync_copy(rows.at[pl.ds(0, nrow)],
                            out_hbm.at[pl.ds(lo + s * 640 + off, nrow)])
        pltpu.sync_copy(den0_sh.at[pl.ds(s * 640, 640)], dzbuf)
        pltpu.sync_copy(dzbuf, den_hbm.at[pl.ds(lo + s * 640, 640)])
        pltpu.sync_copy(den1_sh.at[pl.ds(s * 640, 640)], dzbuf)
        pltpu.sync_copy(dzbuf, den_hbm.at[pl.ds(NPAD + lo + s * 640, 640)])

        plsc.subcore_barrier()


def _gat_sc(x, ei, ea, W, a_s, a_d, We, a_e, b, emask):
    h = x @ W
    hr = h.reshape(N, 2, 64)
    S = (hr * a_s).sum(-1)
    D = (hr * a_d).sum(-1)
    e = (ea @ We).reshape(-1, 2, 64)
    eal = (e * a_e).sum(-1)

    s_flat = S.T.reshape(-1)
    d_flat = D.T.reshape(-1)
    eal_flat = jnp.zeros((2, E_PAD), jnp.float32).at[:, :E].set(eal.T).reshape(-1)
    src_pad = jnp.zeros((E_PAD,), jnp.int32).at[:E].set(ei[0])
    dst_pad = jnp.zeros((E_PAD,), jnp.int32).at[:E].set(ei[1])

    src2 = src_pad.reshape(EP2, 128)
    dst2 = dst_pad.reshape(EP2, 128)
    eal2 = eal_flat.reshape(2 * EP2, 128)

    al2 = _edge_logit_kernel(src2, dst2, eal2, s_flat, d_flat)

    al = al2.reshape(2, E_PAD)
    emf = jnp.zeros((E_PAD,), jnp.float32).at[:E].set(emask.astype(jnp.float32))
    ex = jnp.exp(al) * emf[None, :]
    ex2 = ex.reshape(2 * EP2, 128)

    out_raw, den_flat = _aggregate_kernel(src2, dst2, ex2[:EP2], ex2[EP2:], h)

    den = jnp.stack([den_flat[:NPAD][:N], den_flat[NPAD:][:N]], axis=1)
    out = out_raw[:N].reshape(N, 2, 64) / (den[:, :, None] + 1e-16)
    return out.reshape(N, 128) + b


def _bn_j(x, g, b, nmask):
    w = nmask.astype(x.dtype)
    cnt = w.sum()
    m = (x * w[:, None]).sum(0) / cnt
    d = x - m
    v = (d * d * w[:, None]).sum(0) / cnt
    return g * (x - m) / jnp.sqrt(v + 1e-5) + b


def _gmp_j(x, bt, nmask):
    w = nmask.astype(x.dtype)
    s = jax.ops.segment_sum(x * w[:, None], bt, num_segments=G)
    c = jax.ops.segment_sum(w, bt, num_segments=G)
    return s / jnp.maximum(c, 1.0)[:, None]


def _pool_j(x, ei, ea, bt, w, nmask, emask):
    score = jnp.tanh((x @ w) / (jnp.linalg.norm(w) + 1e-16))
    Nn = x.shape[0]
    vi = nmask.astype(jnp.int32)
    counts = jax.ops.segment_sum(vi, bt, num_segments=G)
    starts = jnp.concatenate([jnp.zeros((1,), counts.dtype), jnp.cumsum(counts)[:-1]])
    order = jnp.lexsort((-score, bt, 1 - vi))
    bs = bt[order]
    rank = jnp.arange(Nn, dtype=counts.dtype) - starts[bs]
    kper = jnp.ceil(RATIO * counts).astype(counts.dtype)
    keep = (rank < kper[bs]) & nmask[order]
    perm = order[jnp.argsort(~keep)]
    kept = jnp.cumsum(keep)
    new_nmask = jnp.arange(Nn) < kept[-1]
    pos = jnp.where(keep, (kept - 1).astype(ei.dtype), 0)
    new_idx = jnp.zeros((Nn,), dtype=ei.dtype).at[order].set(pos)
    mask_old = jnp.zeros((Nn,), bool).at[order].set(keep)
    new_emask = emask & mask_old[ei[0]] & mask_old[ei[1]]
    new_ei = jnp.where(new_emask[None, :], new_idx[ei], 0)
    new_bt = jnp.where(new_nmask, bt[perm], 0)
    xn = (x[perm] * score[perm][:, None]) * new_nmask[:, None].astype(x.dtype)
    return xn, new_ei, ea, new_bt, new_nmask, new_emask


def _mlp_body(z_ref, fW1_ref, fb1_ref, fW2_ref, fb2_ref, o_ref):
    z = z_ref[...]
    h = jnp.maximum(z @ fW1_ref[...] + fb1_ref[...][None, :], 0.0)
    o_ref[...] = h @ fW2_ref[...] + fb2_ref[...][None, :]


def _mlp_head(z, fW1, fb1, fW2, fb2):
    return pl.pallas_call(
        _mlp_body,
        out_shape=jax.ShapeDtypeStruct((G, 1), jnp.float32),
    )(z, fW1, fb1, fW2, fb2)


def kernel(x, edge_index, edge_attr, batch, W1, as1, ad1, We1, ae1, b1, W2, as2, ad2, We2, ae2, b2, W3, as3, ad3, We3, ae3, b3, g1, be1, g2, be2, g3, be3, pw1, pw2, pw3, fW1, fb1, fW2, fb2):
    h, ei, ea, bt = x, edge_index, edge_attr, batch
    nmask = jnp.ones((x.shape[0],), bool)
    emask = jnp.ones((edge_index.shape[1],), bool)

    h = jax.nn.relu(_bn_j(_gat_sc(h, ei, ea, W1, as1, ad1, We1, ae1, b1, emask), g1, be1, nmask))
    h, ei, ea, bt, nmask, emask = _pool_j(h, ei, ea, bt, pw1, nmask, emask)
    r1 = _gmp_j(h, bt, nmask)

    h = jax.nn.relu(_bn_j(_gat_sc(h, ei, ea, W2, as2, ad2, We2, ae2, b2, emask), g2, be2, nmask))
    h, ei, ea, bt, nmask, emask = _pool_j(h, ei, ea, bt, pw2, nmask, emask)
    r2 = _gmp_j(h, bt, nmask)

    h = jax.nn.relu(_bn_j(_gat_sc(h, ei, ea, W3, as3, ad3, We3, ae3, b3, emask), g3, be3, nmask))
    h, ei, ea, bt, nmask, emask = _pool_j(h, ei, ea, bt, pw3, nmask, emask)
    r3 = _gmp_j(h, bt, nmask)

    z = r1 + r2 + r3
    return _mlp_head(z, fW1, fb1, fW2, fb2)


# speedup vs baseline: 39.5703x; 6.3652x over previous
import dataclasses
import functools

import jax
import jax.numpy as jnp
from jax import lax
from jax.experimental import pallas as pl
from jax.experimental.pallas import tpu as pltpu
from jax.experimental.pallas import tpu_sc as plsc

N = 50000
E = 800000
G = 512
RATIO = 0.9

NC = 2
NS = 16
L = 16

EB = 2048
NB = 25
EPT = EB * NB
E_PAD = NS * EPT

GBR = 256
WCAP = 288
CHUNK = 10240
NCHUNK = 6
NPAD = NCHUNK * CHUNK
DEN_PAD = NS * 3136

_vmesh = plsc.VectorSubcoreMesh(core_axis_name="c", subcore_axis_name="s")

_sc_params = pltpu.CompilerParams()
if "needs_layout_passes" in pltpu.CompilerParams.__dataclass_fields__:
    _sc_params = dataclasses.replace(_sc_params, needs_layout_passes=False)


def _splat(val, n=L):
    return jnp.broadcast_to(val, (n,))


EBR = EB // 128
RPT = EPT // 128
EP2 = E_PAD // 128


@functools.partial(
    pl.kernel,
    out_type=jax.ShapeDtypeStruct((2 * EP2, 128), jnp.float32),
    mesh=_vmesh,
    compiler_params=_sc_params,
    scratch_types=[
        pltpu.VMEM((N,), jnp.float32),
        pltpu.VMEM((N,), jnp.float32),
        pltpu.VMEM((EBR, 128), jnp.int32),
        pltpu.VMEM((EBR, 128), jnp.int32),
        pltpu.VMEM((EBR, 128), jnp.float32),
        pltpu.VMEM((EBR, 128), jnp.float32),
        pltpu.SemaphoreType.DMA,
        pltpu.SemaphoreType.DMA,
        pltpu.SemaphoreType.DMA,
        pltpu.SemaphoreType.DMA,
    ],
)
def _edge_logit_kernel(src_hbm, dst_hbm, eal_hbm, s_hbm, d_hbm,
                       al_hbm,
                       s_vm, d_vm, srcb, dstb, ealb, alb,
                       sem0, sem1, sem2, sem3):
    c = lax.axis_index("c")
    s = lax.axis_index("s")

    cp_s = pltpu.async_copy(s_hbm.at[pl.ds(c * N, N)], s_vm, sem0)
    cp_d = pltpu.async_copy(d_hbm.at[pl.ds(c * N, N)], d_vm, sem1)
    cp_s.wait()
    cp_d.wait()

    def batch(b):
        rbase = s * RPT + b * EBR
        c1 = pltpu.async_copy(src_hbm.at[pl.ds(rbase, EBR)], srcb, sem0)
        c2 = pltpu.async_copy(dst_hbm.at[pl.ds(rbase, EBR)], dstb, sem1)
        c3 = pltpu.async_copy(eal_hbm.at[pl.ds(c * EP2 + rbase, EBR)], ealb, sem2)
        c1.wait(); c2.wait(); c3.wait()

        @pl.loop(0, EBR)
        def _(jr):
            @pl.loop(0, 128, step=L)
            def _(jc):
                sl = pl.ds(jc, L)
                sv = plsc.load_gather(s_vm, [srcb[jr, sl]])
                dv = plsc.load_gather(d_vm, [dstb[jr, sl]])
                al = sv + dv + ealb[jr, sl]
                alb[jr, sl] = jnp.maximum(al, 0.2 * al)

        pltpu.async_copy(alb, al_hbm.at[pl.ds(c * EP2 + rbase, EBR)], sem3).wait()

    @pl.loop(0, NB)
    def _(b):
        batch(b)


@functools.partial(
    pl.kernel,
    out_type=[
        jax.ShapeDtypeStruct((NPAD, 128), jnp.float32),
        jax.ShapeDtypeStruct((2 * NPAD,), jnp.float32),
    ],
    mesh=_vmesh,
    compiler_params=_sc_params,
    scratch_types=[
        pltpu.VMEM((EBR, 128), jnp.int32),
        pltpu.VMEM((EBR, 128), jnp.int32),
        pltpu.VMEM((EBR, 128), jnp.float32),
        pltpu.VMEM((EBR, 128), jnp.float32),
        pltpu.VMEM((WCAP,), jnp.int32),
        pltpu.VMEM((WCAP,), jnp.int32),
        pltpu.VMEM((WCAP,), jnp.float32),
        pltpu.VMEM((WCAP,), jnp.float32),
        pltpu.VMEM((2, 128), jnp.int32),
        pltpu.VMEM((2, 128), jnp.int32),
        pltpu.VMEM((2, 128), jnp.float32),
        pltpu.VMEM((2, 128), jnp.float32),
        pltpu.VMEM((GBR, 128), jnp.float32),
        pltpu.VMEM((640,), jnp.float32),
        pltpu.SMEM((1,), jnp.int32),
        pltpu.VMEM_SHARED((CHUNK, 128), jnp.float32),
        pltpu.VMEM_SHARED((CHUNK,), jnp.float32),
        pltpu.VMEM_SHARED((CHUNK,), jnp.float32),
        pltpu.SemaphoreType.DMA,
        pltpu.SemaphoreType.DMA,
        pltpu.SemaphoreType.DMA,
        pltpu.SemaphoreType.DMA,
        pltpu.SemaphoreType.DMA,
    ],
)
def _aggregate_kernel(src_hbm, dst_hbm, ex0_hbm, ex1_hbm, h_hbm,
                      out_hbm, den_hbm,
                      srcb, dstb, e0b, e1b, wsrc, wdst, ww0, ww1,
                      gsrc, gdst, gw0, gw1, rows, dzbuf, cnt_ref,
                      acc_sh, den0_sh, den1_sh,
                      sem0, sem1, sem2, sem3, semg):
    c = lax.axis_index("c")
    s = lax.axis_index("s")

    def drain():
        for j in range(2):
            @pl.loop(0, 128, step=L)
            def _(i):
                gsrc[j, pl.ds(i, L)] = wsrc[pl.ds(j * 128 + i, L)]
                gdst[j, pl.ds(i, L)] = wdst[pl.ds(j * 128 + i, L)]
                gw0[j, pl.ds(i, L)] = ww0[pl.ds(j * 128 + i, L)]
                gw1[j, pl.ds(i, L)] = ww1[pl.ds(j * 128 + i, L)]

        cps = [pltpu.async_copy(h_hbm.at[gsrc.at[j]],
                                rows.at[pl.ds(j * 128, 128)], semg)
               for j in range(2)]
        for cp in cps:
            cp.wait()

        @pl.loop(0, GBR)
        def _(r):
            ridx = _splat(r)
            w0 = plsc.load_gather(ww0, [ridx])
            w1 = plsc.load_gather(ww1, [ridx])
            for cc in range(4):
                sl = pl.ds(cc * L, L)
                rows[r, sl] = rows[r, sl] * w0
            for cc in range(4, 8):
                sl = pl.ds(cc * L, L)
                rows[r, sl] = rows[r, sl] * w1

        for j in range(2):
            pltpu.sync_copy(rows.at[pl.ds(j * 128, 128)],
                            acc_sh.at[gdst.at[j]], add=True)
            pltpu.sync_copy(gw0.at[j], den0_sh.at[gdst.at[j]], add=True)
            pltpu.sync_copy(gw1.at[j], den1_sh.at[gdst.at[j]], add=True)
        cnt = cnt_ref[0]
        wsrc[pl.ds(0, L)] = wsrc[pl.ds(GBR, L)]
        wdst[pl.ds(0, L)] = wdst[pl.ds(GBR, L)]
        ww0[pl.ds(0, L)] = ww0[pl.ds(GBR, L)]
        ww1[pl.ds(0, L)] = ww1[pl.ds(GBR, L)]
        cnt_ref[0] = jnp.maximum(cnt - GBR, 0)

    for p in range(3):
        chunk = 2 * p + c
        lo = chunk * CHUNK

        @pl.loop(0, GBR)
        def _(r):
            @pl.loop(0, 128, step=L)
            def _(j):
                rows[r, pl.ds(j, L)] = jnp.zeros((L,), jnp.float32)

        for off, nrow in ((0, 256), (256, 256), (512, 128)):
            pltpu.sync_copy(rows.at[pl.ds(0, nrow)],
                            acc_sh.at[pl.ds(s * 640 + off, nrow)])
        for k in range(5):
            pltpu.sync_copy(rows.at[0], den0_sh.at[pl.ds(s * 640 + k * 128, 128)])
            pltpu.sync_copy(rows.at[0], den1_sh.at[pl.ds(s * 640 + k * 128, 128)])

        @pl.loop(0, WCAP, step=L)
        def _(i):
            wsrc[pl.ds(i, L)] = jnp.zeros((L,), jnp.int32)
            wdst[pl.ds(i, L)] = jnp.zeros((L,), jnp.int32)
            ww0[pl.ds(i, L)] = jnp.zeros((L,), jnp.float32)
            ww1[pl.ds(i, L)] = jnp.zeros((L,), jnp.float32)

        cnt_ref[0] = 0
        plsc.subcore_barrier()

        def batch(b):
            rbase = s * RPT + b * EBR
            c1 = pltpu.async_copy(src_hbm.at[pl.ds(rbase, EBR)], srcb, sem0)
            c2 = pltpu.async_copy(dst_hbm.at[pl.ds(rbase, EBR)], dstb, sem1)
            c3 = pltpu.async_copy(ex0_hbm.at[pl.ds(rbase, EBR)], e0b, sem2)
            c4 = pltpu.async_copy(ex1_hbm.at[pl.ds(rbase, EBR)], e1b, sem3)
            c1.wait(); c2.wait(); c3.wait(); c4.wait()

            @pl.loop(0, EBR)
            def _(jr):
                @pl.loop(0, 128, step=L)
                def _(jc):
                    sl = pl.ds(jc, L)
                    dv = dstb[jr, sl]
                    dl = dv - lo
                    w0 = e0b[jr, sl]
                    w1 = e1b[jr, sl]
                    m = (dl >= 0) & (dl < CHUNK) & ((w0 + w1) > 0.0)
                    cnt = cnt_ref[0]
                    plsc.store_compressed(wsrc.at[pl.ds(cnt, L)], srcb[jr, sl], mask=m)
                    plsc.store_compressed(wdst.at[pl.ds(cnt, L)], dl, mask=m)
                    plsc.store_compressed(ww0.at[pl.ds(cnt, L)], w0, mask=m)
                    plsc.store_compressed(ww1.at[pl.ds(cnt, L)], w1, mask=m)
                    cnt_ref[0] = cnt + jnp.sum(m.astype(jnp.int32))

                    @pl.when(cnt_ref[0] >= GBR)
                    def _():
                        drain()

        @pl.loop(0, NB)
        def _(b):
            batch(b)

        cnt_tail = cnt_ref[0]

        @pl.loop(0, WCAP, step=L)
        def _(i):
            lane = lax.iota(jnp.int32, L) + i
            dead = lane >= cnt_tail
            ww0[pl.ds(i, L)] = jnp.where(dead, 0.0, ww0[pl.ds(i, L)])
            ww1[pl.ds(i, L)] = jnp.where(dead, 0.0, ww1[pl.ds(i, L)])

        drain()
        cnt_ref[0] = 0
        plsc.subcore_barrier()

        for off, nrow in ((0, 256), (256, 256), (512, 128)):
            pltpu.sync_copy(acc_sh.at[pl.ds(s * 640 + off, nrow)],
                            rows.at[pl.ds(0, nrow)])
            pltpu.sync_copy(rows.at[pl.ds(0, nrow)],
                            out_hbm.at[pl.ds(lo + s * 640 + off, nrow)])
        pltpu.sync_copy(den0_sh.at[pl.ds(s * 640, 640)], dzbuf)
        pltpu.sync_copy(dzbuf, den_hbm.at[pl.ds(lo + s * 640, 640)])
        pltpu.sync_copy(den1_sh.at[pl.ds(s * 640, 640)], dzbuf)
        pltpu.sync_copy(dzbuf, den_hbm.at[pl.ds(NPAD + lo + s * 640, 640)])

        plsc.subcore_barrier()


def _gat_sc(x, src2, dst2, ea, W, a_s, a_d, We, a_e, b, valid):
    h = x @ W
    hr = h.reshape(N, 2, 64)
    S = (hr * a_s).sum(-1)
    D = (hr * a_d).sum(-1)
    S = jnp.where(valid[:, None] > 0, S, -1e30)
    D = jnp.where(valid[:, None] > 0, D, -1e30)
    e = (ea @ We).reshape(-1, 2, 64)
    eal = (e * a_e).sum(-1)

    s_flat = S.T.reshape(-1)
    d_flat = D.T.reshape(-1)
    eal2 = jnp.full((2, E_PAD), -1e30, jnp.float32).at[:, :E].set(eal.T) \
        .reshape(2 * EP2, 128)

    al2 = _edge_logit_kernel(src2, dst2, eal2, s_flat, d_flat)

    ex2 = jnp.exp(al2)

    out_raw, den_flat = _aggregate_kernel(src2, dst2, ex2[:EP2], ex2[EP2:], h)

    den = jnp.stack([den_flat[:NPAD][:N], den_flat[NPAD:][:N]], axis=1)
    out = out_raw[:N].reshape(N, 2, 64) / (den[:, :, None] + 1e-16)
    return out.reshape(N, 128) + b


def _bn_j(x, g, b, nmask):
    w = nmask.astype(x.dtype)
    cnt = w.sum()
    m = (x * w[:, None]).sum(0) / cnt
    d = x - m
    v = (d * d * w[:, None]).sum(0) / cnt
    return g * (x - m) / jnp.sqrt(v + 1e-5) + b


def _gmp_j(x, bt, nmask):
    w = nmask.astype(x.dtype)
    s = jax.ops.segment_sum(x * w[:, None], bt, num_segments=G)
    c = jax.ops.segment_sum(w, bt, num_segments=G)
    return s / jnp.maximum(c, 1.0)[:, None]


MPAD = 256


def _mlp_body(z_ref, fW1_ref, fb1_ref, fW2_ref, fb2_ref, o_ref):
    z = z_ref[...]
    h = jnp.maximum(z @ fW1_ref[...] + fb1_ref[...][None, :], 0.0)
    o_ref[...] = h @ fW2_ref[...] + fb2_ref[...][None, :]


def _mlp_head(z, fW1, fb1, fW2, fb2):
    return pl.pallas_call(
        _mlp_body,
        out_shape=jax.ShapeDtypeStruct((G, 1), jnp.float32),
    )(z, fW1, fb1, fW2, fb2)


def kernel(x, edge_index, edge_attr, batch, W1, as1, ad1, We1, ae1, b1, W2, as2, ad2, We2, ae2, b2, W3, as3, ad3, We3, ae3, b3, g1, be1, g2, be2, g3, be3, pw1, pw2, pw3, fW1, fb1, fW2, fb2):
    bt = batch
    ea = edge_attr

    src_pad = jnp.zeros((E_PAD,), jnp.int32).at[:E].set(edge_index[0])
    dst_pad = jnp.zeros((E_PAD,), jnp.int32).at[:E].set(edge_index[1])
    src2 = src_pad.reshape(EP2, 128)
    dst2 = dst_pad.reshape(EP2, 128)

    bounds = jnp.searchsorted(bt, jnp.arange(G + 1, dtype=bt.dtype), side="left")
    starts0 = bounds[:-1]
    counts = (bounds[1:] - bounds[:-1]).astype(jnp.int32)
    jj = jnp.arange(MPAD, dtype=jnp.int32)
    idx_pad = jnp.minimum(starts0[:, None] + jj[None, :], N - 1)
    jvalid = jj[None, :] < counts[:, None]

    valid = jnp.ones((N,), jnp.float32)
    h = x
    rs = []
    layers = (
        (W1, as1, ad1, We1, ae1, b1, g1, be1, pw1),
        (W2, as2, ad2, We2, ae2, b2, g2, be2, pw2),
        (W3, as3, ad3, We3, ae3, b3, g3, be3, pw3),
    )
    for (W, a_s, a_d, We, a_e, bb, g, be, pw) in layers:
        y = _gat_sc(h, src2, dst2, ea, W, a_s, a_d, We, a_e, bb, valid)
        y = jax.nn.relu(_bn_j(y, g, be, valid > 0))

        score = jnp.tanh((y @ pw) / (jnp.linalg.norm(pw) + 1e-16))
        sm = jnp.where(valid > 0, score, -2.0)
        padded = jnp.where(jvalid, sm[idx_pad], -2.0)
        kper = jnp.ceil(RATIO * counts).astype(counts.dtype)
        topv = jax.lax.top_k(padded, MPAD)[0]
        tau = topv[jnp.arange(G), jnp.maximum(kper - 1, 0)]
        keep = (valid > 0) & (score >= tau[bt])
        h = y * score[:, None] * keep[:, None].astype(jnp.float32)
        valid = keep.astype(jnp.float32)
        counts = kper

        csum = jnp.concatenate([jnp.zeros((1, 128), jnp.float32),
                                jnp.cumsum(h, axis=0)], axis=0)
        seg = csum[bounds[1:]] - csum[bounds[:-1]]
        rs.append(seg / jnp.maximum(counts.astype(jnp.float32), 1.0)[:, None])

    z = rs[0] + rs[1] + rs[2]
    return _mlp_head(z, fW1, fb1, fW2, fb2)


# speedup vs baseline: 40.9587x; 1.0351x over previous
import dataclasses
import functools

import jax
import jax.numpy as jnp
from jax import lax
from jax.experimental import pallas as pl
from jax.experimental.pallas import tpu as pltpu
from jax.experimental.pallas import tpu_sc as plsc

N = 50000
E = 800000
G = 512
RATIO = 0.9

NC = 2
NS = 16
L = 16

EB = 2048
NB = 26
EPT = EB * NB
E_PAD = NS * EPT

GBR = 256
WCAP = 288
CHUNK = 8704
NCHUNK = 6
NPAD = NCHUNK * CHUNK
DEN_PAD = NS * 3136

_vmesh = plsc.VectorSubcoreMesh(core_axis_name="c", subcore_axis_name="s")

_sc_params = pltpu.CompilerParams()
if "needs_layout_passes" in pltpu.CompilerParams.__dataclass_fields__:
    _sc_params = dataclasses.replace(_sc_params, needs_layout_passes=False)


def _splat(val, n=L):
    return jnp.broadcast_to(val, (n,))


EBR = EB // 128
RPT = EPT // 128
EP2 = E_PAD // 128


@functools.partial(
    pl.kernel,
    out_type=jax.ShapeDtypeStruct((2 * EP2, 128), jnp.float32),
    mesh=_vmesh,
    compiler_params=_sc_params,
    scratch_types=[
        pltpu.VMEM((N,), jnp.float32),
        pltpu.VMEM((N,), jnp.float32),
        pltpu.VMEM((EBR, 128), jnp.int32),
        pltpu.VMEM((EBR, 128), jnp.int32),
        pltpu.VMEM((EBR, 128), jnp.float32),
        pltpu.VMEM((EBR, 128), jnp.float32),
        pltpu.SemaphoreType.DMA,
        pltpu.SemaphoreType.DMA,
        pltpu.SemaphoreType.DMA,
        pltpu.SemaphoreType.DMA,
    ],
)
def _edge_logit_kernel(src_hbm, dst_hbm, eal_hbm, s_hbm, d_hbm,
                       al_hbm,
                       s_vm, d_vm, srcb, dstb, ealb, alb,
                       sem0, sem1, sem2, sem3):
    c = lax.axis_index("c")
    s = lax.axis_index("s")

    cp_s = pltpu.async_copy(s_hbm.at[pl.ds(c * N, N)], s_vm, sem0)
    cp_d = pltpu.async_copy(d_hbm.at[pl.ds(c * N, N)], d_vm, sem1)
    cp_s.wait()
    cp_d.wait()

    def batch(b):
        rbase = s * RPT + b * EBR
        c1 = pltpu.async_copy(src_hbm.at[pl.ds(rbase, EBR)], srcb, sem0)
        c2 = pltpu.async_copy(dst_hbm.at[pl.ds(rbase, EBR)], dstb, sem1)
        c3 = pltpu.async_copy(eal_hbm.at[pl.ds(c * EP2 + rbase, EBR)], ealb, sem2)
        c1.wait(); c2.wait(); c3.wait()

        @pl.loop(0, EBR)
        def _(jr):
            @pl.loop(0, 128, step=L)
            def _(jc):
                sl = pl.ds(jc, L)
                sv = plsc.load_gather(s_vm, [srcb[jr, sl]])
                dv = plsc.load_gather(d_vm, [dstb[jr, sl]])
                al = sv + dv + ealb[jr, sl]
                alb[jr, sl] = jnp.maximum(al, 0.2 * al)

        pltpu.async_copy(alb, al_hbm.at[pl.ds(c * EP2 + rbase, EBR)], sem3).wait()

    @pl.loop(0, NB)
    def _(b):
        batch(b)


@functools.partial(
    pl.kernel,
    out_type=[
        jax.ShapeDtypeStruct((NPAD, 128), jnp.float32),
        jax.ShapeDtypeStruct((2 * NS * NPAD,), jnp.float32),
    ],
    mesh=_vmesh,
    compiler_params=_sc_params,
    scratch_types=[
        pltpu.VMEM((EBR, 128), jnp.int32),
        pltpu.VMEM((EBR, 128), jnp.int32),
        pltpu.VMEM((EBR, 128), jnp.float32),
        pltpu.VMEM((EBR, 128), jnp.float32),
        pltpu.VMEM((WCAP,), jnp.int32),
        pltpu.VMEM((WCAP,), jnp.int32),
        pltpu.VMEM((WCAP,), jnp.float32),
        pltpu.VMEM((WCAP,), jnp.float32),
        pltpu.VMEM((2, 128), jnp.int32),
        pltpu.VMEM((2, 128), jnp.int32),
        pltpu.VMEM((GBR, 128), jnp.float32),
        pltpu.VMEM((CHUNK,), jnp.float32),
        pltpu.VMEM((CHUNK,), jnp.float32),
        pltpu.SMEM((1,), jnp.int32),
        pltpu.VMEM_SHARED((CHUNK, 128), jnp.float32),
        pltpu.SemaphoreType.DMA,
        pltpu.SemaphoreType.DMA,
        pltpu.SemaphoreType.DMA,
        pltpu.SemaphoreType.DMA,
        pltpu.SemaphoreType.DMA,
    ],
)
def _aggregate_kernel(src_hbm, dst_hbm, ex0_hbm, ex1_hbm, h_hbm,
                      out_hbm, den_hbm,
                      srcb, dstb, e0b, e1b, wsrc, wdst, ww0, ww1,
                      gsrc, gdst, rows, den0_vm, den1_vm, cnt_ref,
                      acc_sh,
                      sem0, sem1, sem2, sem3, semg):
    c = lax.axis_index("c")
    s = lax.axis_index("s")

    def drain():
        for j in range(2):
            @pl.loop(0, 128, step=L)
            def _(i):
                gsrc[j, pl.ds(i, L)] = wsrc[pl.ds(j * 128 + i, L)]
                gdst[j, pl.ds(i, L)] = wdst[pl.ds(j * 128 + i, L)]

        cps = [pltpu.async_copy(h_hbm.at[gsrc.at[j]],
                                rows.at[pl.ds(j * 128, 128)], semg)
               for j in range(2)]
        for cp in cps:
            cp.wait()

        @pl.loop(0, GBR)
        def _(r):
            ridx = _splat(r)
            w0 = plsc.load_gather(ww0, [ridx])
            w1 = plsc.load_gather(ww1, [ridx])
            for cc in range(4):
                sl = pl.ds(cc * L, L)
                rows[r, sl] = rows[r, sl] * w0
            for cc in range(4, 8):
                sl = pl.ds(cc * L, L)
                rows[r, sl] = rows[r, sl] * w1

        adds = [pltpu.async_copy(rows.at[pl.ds(j * 128, 128)],
                                 acc_sh.at[gdst.at[j]], semg, add=True)
                for j in range(2)]
        for cp in adds:
            cp.wait()
        cnt = cnt_ref[0]
        wsrc[pl.ds(0, L)] = wsrc[pl.ds(GBR, L)]
        wdst[pl.ds(0, L)] = wdst[pl.ds(GBR, L)]
        ww0[pl.ds(0, L)] = ww0[pl.ds(GBR, L)]
        ww1[pl.ds(0, L)] = ww1[pl.ds(GBR, L)]
        cnt_ref[0] = jnp.maximum(cnt - GBR, 0)

    for p in range(3):
        chunk = 2 * p + c
        lo = chunk * CHUNK

        @pl.loop(0, GBR)
        def _(r):
            @pl.loop(0, 128, step=L)
            def _(j):
                rows[r, pl.ds(j, L)] = jnp.zeros((L,), jnp.float32)

        for off, nrow in ((0, 256), (256, 256), (512, 32)):
            pltpu.sync_copy(rows.at[pl.ds(0, nrow)],
                            acc_sh.at[pl.ds(s * 544 + off, nrow)])

        @pl.loop(0, CHUNK, step=L)
        def _(i):
            den0_vm[pl.ds(i, L)] = jnp.zeros((L,), jnp.float32)
            den1_vm[pl.ds(i, L)] = jnp.zeros((L,), jnp.float32)

        @pl.loop(0, WCAP, step=L)
        def _(i):
            wsrc[pl.ds(i, L)] = jnp.zeros((L,), jnp.int32)
            wdst[pl.ds(i, L)] = jnp.zeros((L,), jnp.int32)
            ww0[pl.ds(i, L)] = jnp.zeros((L,), jnp.float32)
            ww1[pl.ds(i, L)] = jnp.zeros((L,), jnp.float32)

        cnt_ref[0] = 0
        plsc.subcore_barrier()

        def batch(b):
            rbase = s * RPT + b * EBR
            c1 = pltpu.async_copy(src_hbm.at[pl.ds(rbase, EBR)], srcb, sem0)
            c2 = pltpu.async_copy(dst_hbm.at[pl.ds(rbase, EBR)], dstb, sem1)
            c3 = pltpu.async_copy(ex0_hbm.at[pl.ds(rbase, EBR)], e0b, sem2)
            c4 = pltpu.async_copy(ex1_hbm.at[pl.ds(rbase, EBR)], e1b, sem3)
            c1.wait(); c2.wait(); c3.wait(); c4.wait()

            @pl.loop(0, EBR)
            def _(jr):
                @pl.loop(0, 128, step=L)
                def _(jc):
                    sl = pl.ds(jc, L)
                    dv = dstb[jr, sl]
                    dl = dv - lo
                    w0 = e0b[jr, sl]
                    w1 = e1b[jr, sl]
                    m = (dl >= 0) & (dl < CHUNK) & ((w0 + w1) > 0.0)
                    plsc.addupdate_scatter(den0_vm, [dl], w0, mask=m)
                    plsc.addupdate_scatter(den1_vm, [dl], w1, mask=m)
                    cnt = cnt_ref[0]
                    plsc.store_compressed(wsrc.at[pl.ds(cnt, L)], srcb[jr, sl], mask=m)
                    plsc.store_compressed(wdst.at[pl.ds(cnt, L)], dl, mask=m)
                    plsc.store_compressed(ww0.at[pl.ds(cnt, L)], w0, mask=m)
                    plsc.store_compressed(ww1.at[pl.ds(cnt, L)], w1, mask=m)
                    cnt_ref[0] = cnt + jnp.sum(m.astype(jnp.int32))

                    @pl.when(cnt_ref[0] >= GBR)
                    def _():
                        drain()

        @pl.loop(0, NB)
        def _(b):
            batch(b)

        cnt_tail = cnt_ref[0]

        @pl.loop(0, WCAP, step=L)
        def _(i):
            lane = lax.iota(jnp.int32, L) + i
            dead = lane >= cnt_tail
            ww0[pl.ds(i, L)] = jnp.where(dead, 0.0, ww0[pl.ds(i, L)])
            ww1[pl.ds(i, L)] = jnp.where(dead, 0.0, ww1[pl.ds(i, L)])

        drain()
        cnt_ref[0] = 0
        plsc.subcore_barrier()

        for off, nrow in ((0, 256), (256, 256), (512, 32)):
            pltpu.sync_copy(acc_sh.at[pl.ds(s * 544 + off, nrow)],
                            rows.at[pl.ds(0, nrow)])
            pltpu.sync_copy(rows.at[pl.ds(0, nrow)],
                            out_hbm.at[pl.ds(lo + s * 544 + off, nrow)])
        d0 = pltpu.async_copy(
            den0_vm, den_hbm.at[pl.ds(s * NPAD + lo, CHUNK)], sem0)
        d1 = pltpu.async_copy(
            den1_vm, den_hbm.at[pl.ds(NS * NPAD + s * NPAD + lo, CHUNK)], sem1)
        d0.wait()
        d1.wait()

        plsc.subcore_barrier()


def _gat_sc(x, src2, dst2, ea, W, a_s, a_d, We, a_e, b, valid):
    h = x @ W
    hr = h.reshape(N, 2, 64)
    S = (hr * a_s).sum(-1)
    D = (hr * a_d).sum(-1)
    S = jnp.where(valid[:, None] > 0, S, -1e30)
    D = jnp.where(valid[:, None] > 0, D, -1e30)
    e = (ea @ We).reshape(-1, 2, 64)
    eal = (e * a_e).sum(-1)

    s_flat = S.T.reshape(-1)
    d_flat = D.T.reshape(-1)
    eal2 = jnp.full((2, E_PAD), -1e30, jnp.float32).at[:, :E].set(eal.T) \
        .reshape(2 * EP2, 128)

    al2 = _edge_logit_kernel(src2, dst2, eal2, s_flat, d_flat)

    ex2 = jnp.exp(al2)

    out_raw, den_flat = _aggregate_kernel(src2, dst2, ex2[:EP2], ex2[EP2:], h)

    denp = den_flat.reshape(2, NS, NPAD).sum(axis=1)
    den = jnp.stack([denp[0, :N], denp[1, :N]], axis=1)
    out = out_raw[:N].reshape(N, 2, 64) / (den[:, :, None] + 1e-16)
    return out.reshape(N, 128) + b


def _bn_j(x, g, b, nmask):
    w = nmask.astype(x.dtype)
    cnt = w.sum()
    m = (x * w[:, None]).sum(0) / cnt
    d = x - m
    v = (d * d * w[:, None]).sum(0) / cnt
    return g * (x - m) / jnp.sqrt(v + 1e-5) + b


def _gmp_j(x, bt, nmask):
    w = nmask.astype(x.dtype)
    s = jax.ops.segment_sum(x * w[:, None], bt, num_segments=G)
    c = jax.ops.segment_sum(w, bt, num_segments=G)
    return s / jnp.maximum(c, 1.0)[:, None]


MPAD = 256


def _mlp_body(z_ref, fW1_ref, fb1_ref, fW2_ref, fb2_ref, o_ref):
    z = z_ref[...]
    h = jnp.maximum(z @ fW1_ref[...] + fb1_ref[...][None, :], 0.0)
    o_ref[...] = h @ fW2_ref[...] + fb2_ref[...][None, :]


def _mlp_head(z, fW1, fb1, fW2, fb2):
    return pl.pallas_call(
        _mlp_body,
        out_shape=jax.ShapeDtypeStruct((G, 1), jnp.float32),
    )(z, fW1, fb1, fW2, fb2)


def kernel(x, edge_index, edge_attr, batch, W1, as1, ad1, We1, ae1, b1, W2, as2, ad2, We2, ae2, b2, W3, as3, ad3, We3, ae3, b3, g1, be1, g2, be2, g3, be3, pw1, pw2, pw3, fW1, fb1, fW2, fb2):
    bt = batch
    ea = edge_attr

    src_pad = jnp.zeros((E_PAD,), jnp.int32).at[:E].set(edge_index[0])
    dst_pad = jnp.zeros((E_PAD,), jnp.int32).at[:E].set(edge_index[1])
    src2 = src_pad.reshape(EP2, 128)
    dst2 = dst_pad.reshape(EP2, 128)

    bounds = jnp.searchsorted(bt, jnp.arange(G + 1, dtype=bt.dtype), side="left")
    starts0 = bounds[:-1]
    counts = (bounds[1:] - bounds[:-1]).astype(jnp.int32)
    jj = jnp.arange(MPAD, dtype=jnp.int32)
    idx_pad = jnp.minimum(starts0[:, None] + jj[None, :], N - 1)
    jvalid = jj[None, :] < counts[:, None]

    valid = jnp.ones((N,), jnp.float32)
    h = x
    rs = []
    layers = (
        (W1, as1, ad1, We1, ae1, b1, g1, be1, pw1),
        (W2, as2, ad2, We2, ae2, b2, g2, be2, pw2),
        (W3, as3, ad3, We3, ae3, b3, g3, be3, pw3),
    )
    for (W, a_s, a_d, We, a_e, bb, g, be, pw) in layers:
        y = _gat_sc(h, src2, dst2, ea, W, a_s, a_d, We, a_e, bb, valid)
        y = jax.nn.relu(_bn_j(y, g, be, valid > 0))

        score = jnp.tanh((y @ pw) / (jnp.linalg.norm(pw) + 1e-16))
        sm = jnp.where(valid > 0, score, -2.0)
        padded = jnp.where(jvalid, sm[idx_pad], -2.0)
        kper = jnp.ceil(RATIO * counts).astype(counts.dtype)
        topv = jax.lax.top_k(padded, MPAD)[0]
        tau = topv[jnp.arange(G), jnp.maximum(kper - 1, 0)]
        keep = (valid > 0) & (score >= tau[bt])
        h = y * score[:, None] * keep[:, None].astype(jnp.float32)
        valid = keep.astype(jnp.float32)
        counts = kper

        csum = jnp.concatenate([jnp.zeros((1, 128), jnp.float32),
                                jnp.cumsum(h, axis=0)], axis=0)
        seg = csum[bounds[1:]] - csum[bounds[:-1]]
        rs.append(seg / jnp.maximum(counts.astype(jnp.float32), 1.0)[:, None])

    z = rs[0] + rs[1] + rs[2]
    return _mlp_head(z, fW1, fb1, fW2, fb2)


# speedup vs baseline: 41.2620x; 1.0074x over previous
import dataclasses
import functools

import jax
import jax.numpy as jnp
from jax import lax
from jax.experimental import pallas as pl
from jax.experimental.pallas import tpu as pltpu
from jax.experimental.pallas import tpu_sc as plsc

N = 50000
E = 800000
G = 512
RATIO = 0.9

NC = 2
NS = 16
L = 16

EB = 2048
NB = 26
EPT = EB * NB
E_PAD = NS * EPT

GBR = 256
WCAP = 288
CHUNK = 8704
NCHUNK = 6
NPAD = NCHUNK * CHUNK
DEN_PAD = NS * 3136

_vmesh = plsc.VectorSubcoreMesh(core_axis_name="c", subcore_axis_name="s")

_sc_params = pltpu.CompilerParams()
if "needs_layout_passes" in pltpu.CompilerParams.__dataclass_fields__:
    _sc_params = dataclasses.replace(_sc_params, needs_layout_passes=False)


def _splat(val, n=L):
    return jnp.broadcast_to(val, (n,))


EBR = EB // 128
RPT = EPT // 128
EP2 = E_PAD // 128


@functools.partial(
    pl.kernel,
    out_type=jax.ShapeDtypeStruct((2 * EP2, 128), jnp.float32),
    mesh=_vmesh,
    compiler_params=_sc_params,
    scratch_types=[
        pltpu.VMEM((N,), jnp.float32),
        pltpu.VMEM((N,), jnp.float32),
        pltpu.VMEM((EBR, 128), jnp.int32),
        pltpu.VMEM((EBR, 128), jnp.int32),
        pltpu.VMEM((EBR, 128), jnp.float32),
        pltpu.VMEM((EBR, 128), jnp.float32),
        pltpu.SemaphoreType.DMA,
        pltpu.SemaphoreType.DMA,
        pltpu.SemaphoreType.DMA,
        pltpu.SemaphoreType.DMA,
    ],
)
def _edge_logit_kernel(src_hbm, dst_hbm, eal_hbm, s_hbm, d_hbm,
                       al_hbm,
                       s_vm, d_vm, srcb, dstb, ealb, alb,
                       sem0, sem1, sem2, sem3):
    c = lax.axis_index("c")
    s = lax.axis_index("s")

    cp_s = pltpu.async_copy(s_hbm.at[pl.ds(c * N, N)], s_vm, sem0)
    cp_d = pltpu.async_copy(d_hbm.at[pl.ds(c * N, N)], d_vm, sem1)
    cp_s.wait()
    cp_d.wait()

    def batch(b):
        rbase = s * RPT + b * EBR
        c1 = pltpu.async_copy(src_hbm.at[pl.ds(rbase, EBR)], srcb, sem0)
        c2 = pltpu.async_copy(dst_hbm.at[pl.ds(rbase, EBR)], dstb, sem1)
        c3 = pltpu.async_copy(eal_hbm.at[pl.ds(c * EP2 + rbase, EBR)], ealb, sem2)
        c1.wait(); c2.wait(); c3.wait()

        @pl.loop(0, EBR)
        def _(jr):
            @pl.loop(0, 128, step=L)
            def _(jc):
                sl = pl.ds(jc, L)
                sv = plsc.load_gather(s_vm, [srcb[jr, sl]])
                dv = plsc.load_gather(d_vm, [dstb[jr, sl]])
                al = sv + dv + ealb[jr, sl]
                alb[jr, sl] = jnp.maximum(al, 0.2 * al)

        pltpu.async_copy(alb, al_hbm.at[pl.ds(c * EP2 + rbase, EBR)], sem3).wait()

    @pl.loop(0, NB)
    def _(b):
        batch(b)


@functools.partial(
    pl.kernel,
    out_type=[
        jax.ShapeDtypeStruct((NPAD, 128), jnp.float32),
        jax.ShapeDtypeStruct((2 * NS * NPAD,), jnp.float32),
    ],
    mesh=_vmesh,
    compiler_params=_sc_params,
    scratch_types=[
        pltpu.VMEM((EBR, 128), jnp.int32),
        pltpu.VMEM((EBR, 128), jnp.int32),
        pltpu.VMEM((EBR, 128), jnp.float32),
        pltpu.VMEM((EBR, 128), jnp.float32),
        pltpu.VMEM((WCAP,), jnp.int32),
        pltpu.VMEM((WCAP,), jnp.int32),
        pltpu.VMEM((WCAP,), jnp.float32),
        pltpu.VMEM((WCAP,), jnp.float32),
        pltpu.VMEM((2, 128), jnp.int32),
        pltpu.VMEM((2, 128), jnp.int32),
        pltpu.VMEM((GBR, 128), jnp.float32),
        pltpu.VMEM((CHUNK,), jnp.float32),
        pltpu.VMEM((CHUNK,), jnp.float32),
        pltpu.SMEM((1,), jnp.int32),
        pltpu.VMEM_SHARED((CHUNK, 128), jnp.float32),
        pltpu.SemaphoreType.DMA,
        pltpu.SemaphoreType.DMA,
        pltpu.SemaphoreType.DMA,
        pltpu.SemaphoreType.DMA,
        pltpu.SemaphoreType.DMA,
    ],
)
def _aggregate_kernel(src_hbm, dst_hbm, ex0_hbm, ex1_hbm, h_hbm,
                      out_hbm, den_hbm,
                      srcb, dstb, e0b, e1b, wsrc, wdst, ww0, ww1,
                      gsrc, gdst, rows, den0_vm, den1_vm, cnt_ref,
                      acc_sh,
                      sem0, sem1, sem2, sem3, semg):
    c = lax.axis_index("c")
    s = lax.axis_index("s")

    def drain():
        for j in range(2):
            @pl.loop(0, 128, step=L)
            def _(i):
                gsrc[j, pl.ds(i, L)] = wsrc[pl.ds(j * 128 + i, L)]
                gdst[j, pl.ds(i, L)] = wdst[pl.ds(j * 128 + i, L)]

        cps = [pltpu.async_copy(h_hbm.at[gsrc.at[j]],
                                rows.at[pl.ds(j * 128, 128)], semg)
               for j in range(2)]
        for cp in cps:
            cp.wait()

        @pl.loop(0, GBR, unroll=2)
        def _(r):
            ridx = _splat(r)
            w0 = plsc.load_gather(ww0, [ridx])
            w1 = plsc.load_gather(ww1, [ridx])
            for cc in range(4):
                sl = pl.ds(cc * L, L)
                rows[r, sl] = rows[r, sl] * w0
            for cc in range(4, 8):
                sl = pl.ds(cc * L, L)
                rows[r, sl] = rows[r, sl] * w1

        adds = [pltpu.async_copy(rows.at[pl.ds(j * 128, 128)],
                                 acc_sh.at[gdst.at[j]], semg, add=True)
                for j in range(2)]
        for cp in adds:
            cp.wait()
        cnt = cnt_ref[0]
        wsrc[pl.ds(0, L)] = wsrc[pl.ds(GBR, L)]
        wdst[pl.ds(0, L)] = wdst[pl.ds(GBR, L)]
        ww0[pl.ds(0, L)] = ww0[pl.ds(GBR, L)]
        ww1[pl.ds(0, L)] = ww1[pl.ds(GBR, L)]
        cnt_ref[0] = jnp.maximum(cnt - GBR, 0)

    for p in range(3):
        chunk = 2 * p + c
        lo = chunk * CHUNK

        @pl.loop(0, GBR)
        def _(r):
            @pl.loop(0, 128, step=L)
            def _(j):
                rows[r, pl.ds(j, L)] = jnp.zeros((L,), jnp.float32)

        for off, nrow in ((0, 256), (256, 256), (512, 32)):
            pltpu.sync_copy(rows.at[pl.ds(0, nrow)],
                            acc_sh.at[pl.ds(s * 544 + off, nrow)])

        @pl.loop(0, CHUNK, step=L)
        def _(i):
            den0_vm[pl.ds(i, L)] = jnp.zeros((L,), jnp.float32)
            den1_vm[pl.ds(i, L)] = jnp.zeros((L,), jnp.float32)

        @pl.loop(0, WCAP, step=L)
        def _(i):
            wsrc[pl.ds(i, L)] = jnp.zeros((L,), jnp.int32)
            wdst[pl.ds(i, L)] = jnp.zeros((L,), jnp.int32)
            ww0[pl.ds(i, L)] = jnp.zeros((L,), jnp.float32)
            ww1[pl.ds(i, L)] = jnp.zeros((L,), jnp.float32)

        cnt_ref[0] = 0
        plsc.subcore_barrier()

        def batch(b):
            rbase = s * RPT + b * EBR
            c1 = pltpu.async_copy(src_hbm.at[pl.ds(rbase, EBR)], srcb, sem0)
            c2 = pltpu.async_copy(dst_hbm.at[pl.ds(rbase, EBR)], dstb, sem1)
            c3 = pltpu.async_copy(ex0_hbm.at[pl.ds(rbase, EBR)], e0b, sem2)
            c4 = pltpu.async_copy(ex1_hbm.at[pl.ds(rbase, EBR)], e1b, sem3)
            c1.wait(); c2.wait(); c3.wait(); c4.wait()

            @pl.loop(0, EBR)
            def _(jr):
                @pl.loop(0, 128, step=L)
                def _(jc):
                    sl = pl.ds(jc, L)
                    dv = dstb[jr, sl]
                    dl = dv - lo
                    w0 = e0b[jr, sl]
                    w1 = e1b[jr, sl]
                    m = (dl >= 0) & (dl < CHUNK) & ((w0 + w1) > 0.0)
                    plsc.addupdate_scatter(den0_vm, [dl], w0, mask=m)
                    plsc.addupdate_scatter(den1_vm, [dl], w1, mask=m)
                    cnt = cnt_ref[0]
                    plsc.store_compressed(wsrc.at[pl.ds(cnt, L)], srcb[jr, sl], mask=m)
                    plsc.store_compressed(wdst.at[pl.ds(cnt, L)], dl, mask=m)
                    plsc.store_compressed(ww0.at[pl.ds(cnt, L)], w0, mask=m)
                    plsc.store_compressed(ww1.at[pl.ds(cnt, L)], w1, mask=m)
                    cnt_ref[0] = cnt + jnp.sum(m.astype(jnp.int32))

                    @pl.when(cnt_ref[0] >= GBR)
                    def _():
                        drain()

        @pl.loop(0, NB)
        def _(b):
            batch(b)

        cnt_tail = cnt_ref[0]

        @pl.loop(0, WCAP, step=L)
        def _(i):
            lane = lax.iota(jnp.int32, L) + i
            dead = lane >= cnt_tail
            ww0[pl.ds(i, L)] = jnp.where(dead, 0.0, ww0[pl.ds(i, L)])
            ww1[pl.ds(i, L)] = jnp.where(dead, 0.0, ww1[pl.ds(i, L)])

        drain()
        cnt_ref[0] = 0
        plsc.subcore_barrier()

        for off, nrow in ((0, 256), (256, 256), (512, 32)):
            pltpu.sync_copy(acc_sh.at[pl.ds(s * 544 + off, nrow)],
                            rows.at[pl.ds(0, nrow)])
            pltpu.sync_copy(rows.at[pl.ds(0, nrow)],
                            out_hbm.at[pl.ds(lo + s * 544 + off, nrow)])
        d0 = pltpu.async_copy(
            den0_vm, den_hbm.at[pl.ds(s * NPAD + lo, CHUNK)], sem0)
        d1 = pltpu.async_copy(
            den1_vm, den_hbm.at[pl.ds(NS * NPAD + s * NPAD + lo, CHUNK)], sem1)
        d0.wait()
        d1.wait()

        plsc.subcore_barrier()


def _gat_sc(x, src2, dst2, ea, W, a_s, a_d, We, a_e, b, valid):
    h = x @ W
    hr = h.reshape(N, 2, 64)
    S = (hr * a_s).sum(-1)
    D = (hr * a_d).sum(-1)
    S = jnp.where(valid[:, None] > 0, S, -1e30)
    D = jnp.where(valid[:, None] > 0, D, -1e30)
    e = (ea @ We).reshape(-1, 2, 64)
    eal = (e * a_e).sum(-1)

    s_flat = S.T.reshape(-1)
    d_flat = D.T.reshape(-1)
    eal2 = jnp.full((2, E_PAD), -1e30, jnp.float32).at[:, :E].set(eal.T) \
        .reshape(2 * EP2, 128)

    al2 = _edge_logit_kernel(src2, dst2, eal2, s_flat, d_flat)

    ex2 = jnp.exp(al2)

    out_raw, den_flat = _aggregate_kernel(src2, dst2, ex2[:EP2], ex2[EP2:], h)

    denp = den_flat.reshape(2, NS, NPAD).sum(axis=1)
    den = jnp.stack([denp[0, :N], denp[1, :N]], axis=1)
    out = out_raw[:N].reshape(N, 2, 64) / (den[:, :, None] + 1e-16)
    return out.reshape(N, 128) + b


def _bn_j(x, g, b, nmask):
    w = nmask.astype(x.dtype)
    cnt = w.sum()
    m = (x * w[:, None]).sum(0) / cnt
    d = x - m
    v = (d * d * w[:, None]).sum(0) / cnt
    return g * (x - m) / jnp.sqrt(v + 1e-5) + b


def _gmp_j(x, bt, nmask):
    w = nmask.astype(x.dtype)
    s = jax.ops.segment_sum(x * w[:, None], bt, num_segments=G)
    c = jax.ops.segment_sum(w, bt, num_segments=G)
    return s / jnp.maximum(c, 1.0)[:, None]


MPAD = 256


def _mlp_body(z_ref, fW1_ref, fb1_ref, fW2_ref, fb2_ref, o_ref):
    z = z_ref[...]
    h = jnp.maximum(z @ fW1_ref[...] + fb1_ref[...][None, :], 0.0)
    o_ref[...] = h @ fW2_ref[...] + fb2_ref[...][None, :]


def _mlp_head(z, fW1, fb1, fW2, fb2):
    return pl.pallas_call(
        _mlp_body,
        out_shape=jax.ShapeDtypeStruct((G, 1), jnp.float32),
    )(z, fW1, fb1, fW2, fb2)


def kernel(x, edge_index, edge_attr, batch, W1, as1, ad1, We1, ae1, b1, W2, as2, ad2, We2, ae2, b2, W3, as3, ad3, We3, ae3, b3, g1, be1, g2, be2, g3, be3, pw1, pw2, pw3, fW1, fb1, fW2, fb2):
    bt = batch
    ea = edge_attr

    src_pad = jnp.zeros((E_PAD,), jnp.int32).at[:E].set(edge_index[0])
    dst_pad = jnp.zeros((E_PAD,), jnp.int32).at[:E].set(edge_index[1])
    src2 = src_pad.reshape(EP2, 128)
    dst2 = dst_pad.reshape(EP2, 128)

    bounds = jnp.searchsorted(bt, jnp.arange(G + 1, dtype=bt.dtype), side="left")
    starts0 = bounds[:-1]
    counts = (bounds[1:] - bounds[:-1]).astype(jnp.int32)
    jj = jnp.arange(MPAD, dtype=jnp.int32)
    idx_pad = jnp.minimum(starts0[:, None] + jj[None, :], N - 1)
    jvalid = jj[None, :] < counts[:, None]

    valid = jnp.ones((N,), jnp.float32)
    h = x
    rs = []
    layers = (
        (W1, as1, ad1, We1, ae1, b1, g1, be1, pw1),
        (W2, as2, ad2, We2, ae2, b2, g2, be2, pw2),
        (W3, as3, ad3, We3, ae3, b3, g3, be3, pw3),
    )
    for (W, a_s, a_d, We, a_e, bb, g, be, pw) in layers:
        y = _gat_sc(h, src2, dst2, ea, W, a_s, a_d, We, a_e, bb, valid)
        y = jax.nn.relu(_bn_j(y, g, be, valid > 0))

        score = jnp.tanh((y @ pw) / (jnp.linalg.norm(pw) + 1e-16))
        sm = jnp.where(valid > 0, score, -2.0)
        padded = jnp.where(jvalid, sm[idx_pad], -2.0)
        kper = jnp.ceil(RATIO * counts).astype(counts.dtype)
        topv = jax.lax.top_k(padded, MPAD)[0]
        tau = topv[jnp.arange(G), jnp.maximum(kper - 1, 0)]
        keep = (valid > 0) & (score >= tau[bt])
        h = y * score[:, None] * keep[:, None].astype(jnp.float32)
        valid = keep.astype(jnp.float32)
        counts = kper

        csum = jnp.concatenate([jnp.zeros((1, 128), jnp.float32),
                                jnp.cumsum(h, axis=0)], axis=0)
        seg = csum[bounds[1:]] - csum[bounds[:-1]]
        rs.append(seg / jnp.maximum(counts.astype(jnp.float32), 1.0)[:, None])

    z = rs[0] + rs[1] + rs[2]
    return _mlp_head(z, fW1, fb1, fW2, fb2)
